# Initial kernel scaffold; baseline (speedup 1.0000x reference)
import jax
import jax.numpy as jnp
from jax.experimental import pallas as pl


def kernel(gt_points, outputs_coord, outputs_class, outputs_traj):
    raise NotImplementedError("write your pallas kernel here")


# speedup vs baseline: 1342.4458x; 1342.4458x over previous
import functools

import jax
import jax.numpy as jnp
from jax import lax
from jax.experimental import pallas as pl
from jax.experimental.pallas import tpu as pltpu
from jax.experimental.pallas import tpu_sc as plsc

B = 32
N = 20000
F = 30
L = 16
CH = N // L
NEG = -1e30
BIGI = 0x7FFFFFFF
NMS_TH2 = 4.0


def _splat_gather(ref, idx_scalar):
    iv = jnp.broadcast_to(idx_scalar, (L,))
    return plsc.load_gather(ref, [iv])


def _make_sc_kernel():
    mesh = plsc.VectorSubcoreMesh(core_axis_name="c", subcore_axis_name="s")

    @functools.partial(
        pl.kernel,
        mesh=mesh,
        compiler_params=pltpu.CompilerParams(needs_layout_passes=False),
        out_type=[
            jax.ShapeDtypeStruct((B, L), jnp.float32),
            jax.ShapeDtypeStruct((B, 8, 60), jnp.float32),
        ],
        scratch_types=[
            pltpu.VMEM((N,), jnp.float32),
            pltpu.VMEM((N,), jnp.float32),
            pltpu.VMEM((N,), jnp.float32),
            pltpu.VMEM((N,), jnp.float32),
            pltpu.VMEM((L,), jnp.float32),
            pltpu.VMEM((L,), jnp.float32),
            pltpu.VMEM((L,), jnp.float32),
            pltpu.VMEM((8, 60), jnp.float32),
            pltpu.SemaphoreType.DMA,
        ],
    )
    def sc_select(coord_hbm, cls_hbm, glx_hbm, gly_hbm, traj_hbm,
                  info_out, rows_out,
                  xv, yv, cv, sv, glxv, glyv, infov, rowsv, sem):
        b = lax.axis_index("s") * 2 + lax.axis_index("c")
        pltpu.sync_copy(coord_hbm.at[b, 0], xv)
        pltpu.sync_copy(coord_hbm.at[b, 1], yv)
        pltpu.sync_copy(cls_hbm.at[b], cv)
        pltpu.sync_copy(glx_hbm.at[b], glxv)
        pltpu.sync_copy(gly_hbm.at[b], glyv)

        gx = glxv[...]
        gy = glyv[...]
        lanes = lax.broadcasted_iota(jnp.int32, (L,), 0)

        def pass_a(i, c):
            dmin, dmini, smax, smaxi = c
            base = i * L
            xs = xv[pl.ds(base, L)]
            ys = yv[pl.ds(base, L)]
            ss = cv[pl.ds(base, L)]
            dx = xs - gx
            dy = ys - gy
            d2 = dx * dx + dy * dy
            idx = lanes + base
            mlt = d2 < dmin
            dmin = jnp.where(mlt, d2, dmin)
            dmini = jnp.where(mlt, idx, dmini)
            mgt = ss > smax
            smax = jnp.where(mgt, ss, smax)
            smaxi = jnp.where(mgt, idx, smaxi)
            return dmin, dmini, smax, smaxi

        init = (jnp.full((L,), 3.4e38, jnp.float32),
                jnp.zeros((L,), jnp.int32),
                jnp.full((L,), NEG, jnp.float32),
                jnp.zeros((L,), jnp.int32))
        dmin, dmini, smax, smaxi = lax.fori_loop(0, CH, pass_a, init)

        gmin = jnp.min(dmin)
        imin = jnp.min(jnp.where(dmin == gmin, dmini, BIGI))
        g0 = jnp.max(smax)
        sel0 = jnp.min(jnp.where(smax == g0, smaxi, BIGI))

        clsmin = jnp.max(_splat_gather(cv, imin))
        sx = _splat_gather(xv, sel0)
        sy = _splat_gather(yv, sel0)
        de2 = jnp.max((sx - gx) * (sx - gx) + (sy - gy) * (sy - gy))

        gvals = [g0]
        sels = [sel0]
        selx, sely = sx, sy
        for k in range(1, 6):
            src = cv if k == 1 else sv
            write = k < 5

            def nms_pass(i, c, src=src, write=write, selx=selx, sely=sely):
                smax, smaxi = c
                base = i * L
                xs = xv[pl.ds(base, L)]
                ys = yv[pl.ds(base, L)]
                ss = src[pl.ds(base, L)]
                dx = xs - selx
                dy = ys - sely
                d2 = dx * dx + dy * dy
                ss = jnp.where(d2 <= NMS_TH2, NEG, ss)
                if write:
                    sv[pl.ds(base, L)] = ss
                idx = lanes + base
                mgt = ss > smax
                smax = jnp.where(mgt, ss, smax)
                smaxi = jnp.where(mgt, idx, smaxi)
                return smax, smaxi

            smax, smaxi = lax.fori_loop(
                0, CH, nms_pass,
                (jnp.full((L,), NEG, jnp.float32), jnp.zeros((L,), jnp.int32)))
            gk = jnp.max(smax)
            selk = jnp.min(jnp.where(smax == gk, smaxi, BIGI))
            gvals.append(gk)
            sels.append(selk)
            if k < 5:
                selx = _splat_gather(xv, selk)
                sely = _splat_gather(yv, selk)

        founds = [gv > -1e29 for gv in gvals]

        scalars = [clsmin, de2] + gvals + [
            jnp.where(f, 1.0, 0.0).astype(jnp.float32) for f in founds]
        acc = jnp.zeros((L,), jnp.float32)
        for j, v in enumerate(scalars):
            acc = jnp.where(lanes == j, v, acc)
        infov[...] = acc
        pltpu.sync_copy(infov, info_out.at[b])

        row_ids = [imin] + [
            jnp.where(f, s, 0) for f, s in zip(founds, sels)]
        copies = []
        for j, r in enumerate(row_ids):
            copies.append(
                pltpu.async_copy(traj_hbm.at[r + b * N], rowsv.at[j], sem))
        for c in copies:
            c.wait()
        pltpu.sync_copy(rowsv, rows_out.at[b])

    return sc_select


_sc_select = _make_sc_kernel()


def _tc_final(gt_ref, info_ref, rows_ref, loss_ref, de_ref, probs_ref,
              trajs_ref):
    gt = gt_ref[...]
    info = info_ref[...]
    rows = rows_ref[...]
    p = jnp.clip(info[:, 0], 1e-7, 1.0 - 1e-7)
    closs = -jnp.log(p)
    d = rows[:, 0, :] - gt
    ad = jnp.abs(d)
    tl = jnp.mean(jnp.where(ad < 1.0, 0.5 * ad * ad, ad - 0.5), axis=1)
    loss_ref[0, 0] = jnp.mean(tl + closs + 1.0)
    col = lax.broadcasted_iota(jnp.int32, (B, F), 1)
    de_ref[...] = jnp.where(col == F - 1, jnp.sqrt(info[:, 1])[:, None], 0.0)
    kept = info[:, 8:14]
    probs_ref[...] = jnp.where(kept > 0.5, info[:, 2:8], 0.0)
    trajs_ref[...] = rows[:, 1:7, :] * kept[:, :, None]


@jax.jit
def kernel(gt_points, outputs_coord, outputs_class, outputs_traj):
    coord = outputs_coord[:, 0]
    coord_t = jnp.transpose(coord, (0, 2, 1))
    cls2 = outputs_class[:, 0]
    traj2 = outputs_traj.reshape(B * N, F * 2)
    gt_last = gt_points[:, -1, :]
    glx = jnp.broadcast_to(gt_last[:, 0:1], (B, L))
    gly = jnp.broadcast_to(gt_last[:, 1:2], (B, L))

    info, rows = _sc_select(coord_t, cls2, glx, gly, traj2)

    gt2 = gt_points.reshape(B, F * 2)
    loss2, de, probs, trajs = pl.pallas_call(
        _tc_final,
        out_shape=[
            jax.ShapeDtypeStruct((1, 1), jnp.float32),
            jax.ShapeDtypeStruct((B, F), jnp.float32),
            jax.ShapeDtypeStruct((B, 6), jnp.float32),
            jax.ShapeDtypeStruct((B, 6, F * 2), jnp.float32),
        ],
        out_specs=[
            pl.BlockSpec(memory_space=pltpu.SMEM),
            pl.BlockSpec(memory_space=pltpu.VMEM),
            pl.BlockSpec(memory_space=pltpu.VMEM),
            pl.BlockSpec(memory_space=pltpu.VMEM),
        ],
    )(gt2, info, rows)

    loss = loss2[0, 0]
    pred_trajs = trajs.reshape(B, 6, F, 2)
    return loss, de, pred_trajs, probs
